# Initial kernel scaffold; baseline (speedup 1.0000x reference)
import jax
import jax.numpy as jnp
from jax.experimental import pallas as pl


def kernel(x, edge_index, edge_weight, W0, b0, P, W11, b11):
    raise NotImplementedError("write your pallas kernel here")


# speedup vs baseline: 47.1650x; 47.1650x over previous
import functools

import jax
import jax.numpy as jnp
from jax import lax
from jax.experimental import pallas as pl
from jax.experimental.pallas import tpu as pltpu
from jax.experimental.pallas import tpu_sc as plsc

NT = 16
LANE = 16
CHUNK = 2048
ROWS = CHUNK // 128
T_TERMS = 10
N_ROUNDS = 12


def _dense_body(xT_ref, P_ref, W0p_ref, b0r_ref, W11_ref, z0_ref, cw_ref):
    f32 = jnp.float32
    eye = jnp.eye(64, dtype=f32)
    Wp = eye
    for i in range(P_ref.shape[0]):
        Pi = P_ref[i]
        A = Pi - Pi.T
        W = eye
        term = eye
        for k in range(1, T_TERMS + 1):
            term = jnp.dot(term, A, preferred_element_type=f32) * (1.0 / k)
            W = W + term
        Wp = jnp.dot(Wp, W, preferred_element_type=f32)
    u = jnp.dot(Wp, W11_ref[...], preferred_element_type=f32)
    w8 = jnp.dot(W0p_ref[...], u, preferred_element_type=f32)
    c = jnp.dot(b0r_ref[...], u, preferred_element_type=f32)
    z0_ref[...] = jnp.sum(xT_ref[...] * w8, axis=0, keepdims=True)
    cw_ref[...] = jnp.broadcast_to(c, (8, 128))


def _sc_body(nchunk, np_,
             src_hbm, dst_hbm, ew_hbm, z0_hbm, cvec_hbm,
             zout_hbm, norm_hbm,
             table, srcb, dstb, ewb, msgb, slb, zerob, cbuf,
             acc_sh, dinv_sh):
    sl = np_ // NT
    wid = lax.axis_index("s")
    base = wid * sl
    row0w = wid * (nchunk * ROWS)

    def zb(i, _):
        zerob[pl.ds(i * LANE, LANE)] = jnp.zeros((LANE,), jnp.float32)
        return 0
    lax.fori_loop(0, sl // LANE, zb, 0)

    pltpu.sync_copy(zerob, acc_sh.at[pl.ds(base, sl)])
    plsc.subcore_barrier()

    def deg_chunk(j, _):
        r0 = row0w + j * ROWS
        pltpu.sync_copy(dst_hbm.at[pl.ds(r0, ROWS)], dstb)
        pltpu.sync_copy(ew_hbm.at[pl.ds(r0, ROWS)], ewb)
        for r in range(ROWS):
            pltpu.sync_copy(ewb.at[r], acc_sh.at[dstb.at[r]], add=True)
        return 0
    lax.fori_loop(0, nchunk, deg_chunk, 0)
    plsc.subcore_barrier()

    pltpu.sync_copy(acc_sh.at[pl.ds(base, sl)], slb)

    def rsq(i, _):
        d = slb[pl.ds(i * LANE, LANE)]
        dc = jnp.maximum(d, 1.0)
        s = 0.5 * (dc + 1.0)
        for _ in range(12):
            s = 0.5 * (s + dc / s)
        y = 1.0 / s
        for _ in range(3):
            y = y * (1.5 - 0.5 * dc * y * y)
        slb[pl.ds(i * LANE, LANE)] = jnp.where(d > 0.0, y, 0.0)
        return 0
    lax.fori_loop(0, sl // LANE, rsq, 0)
    pltpu.sync_copy(slb, dinv_sh.at[pl.ds(base, sl)])
    plsc.subcore_barrier()
    pltpu.sync_copy(dinv_sh, table)

    def norm_chunk(j, _):
        r0 = row0w + j * ROWS
        pltpu.sync_copy(src_hbm.at[pl.ds(r0, ROWS)], srcb)
        pltpu.sync_copy(dst_hbm.at[pl.ds(r0, ROWS)], dstb)
        pltpu.sync_copy(ew_hbm.at[pl.ds(r0, ROWS)], ewb)

        def nv(v, _):
            r = v >> 3
            q = (v & 7) << 4
            si = srcb[r, pl.ds(q, LANE)]
            di = dstb[r, pl.ds(q, LANE)]
            a = plsc.load_gather(table, [si])
            b = plsc.load_gather(table, [di])
            msgb[r, pl.ds(q, LANE)] = a * ewb[r, pl.ds(q, LANE)] * b
            return 0
        lax.fori_loop(0, CHUNK // LANE, nv, 0)
        pltpu.sync_copy(msgb, norm_hbm.at[pl.ds(r0, ROWS)])
        return 0
    lax.fori_loop(0, nchunk, norm_chunk, 0)

    pltpu.sync_copy(z0_hbm, table)
    pltpu.sync_copy(cvec_hbm, cbuf)

    def round_(it, _):
        pltpu.sync_copy(zerob, acc_sh.at[pl.ds(base, sl)])
        plsc.subcore_barrier()

        def prop_chunk(j, _):
            r0 = row0w + j * ROWS
            pltpu.sync_copy(src_hbm.at[pl.ds(r0, ROWS)], srcb)
            pltpu.sync_copy(dst_hbm.at[pl.ds(r0, ROWS)], dstb)
            pltpu.sync_copy(norm_hbm.at[pl.ds(r0, ROWS)], ewb)

            def gv(v, _):
                r = v >> 3
                q = (v & 7) << 4
                si = srcb[r, pl.ds(q, LANE)]
                zv = plsc.load_gather(table, [si])
                msgb[r, pl.ds(q, LANE)] = zv * ewb[r, pl.ds(q, LANE)]
                return 0
            lax.fori_loop(0, CHUNK // LANE, gv, 0)
            for r in range(ROWS):
                pltpu.sync_copy(msgb.at[r], acc_sh.at[dstb.at[r]], add=True)
            return 0
        lax.fori_loop(0, nchunk, prop_chunk, 0)
        plsc.subcore_barrier()
        pltpu.sync_copy(acc_sh, table)

        @pl.when(it == 0)
        def _add_c():
            cval = cbuf[...]

            def ac(i, _):
                t = table[pl.ds(i * LANE, LANE)]
                table[pl.ds(i * LANE, LANE)] = t + cval
                return 0
            lax.fori_loop(0, np_ // LANE, ac, 0)
        plsc.subcore_barrier()
        return 0
    lax.fori_loop(0, N_ROUNDS, round_, 0)

    pltpu.sync_copy(table.at[pl.ds(base, sl)], zout_hbm.at[pl.ds(base, sl)])


@jax.jit
def kernel(x, edge_index, edge_weight, W0, b0, P, W11, b11):
    f32 = jnp.float32
    n = x.shape[0]
    e = edge_index.shape[1]
    np_ = ((n + 255) // 256) * 256
    et = e + n
    per = NT * CHUNK
    ep = ((et + per - 1) // per) * per
    epr = ep // 128
    nchunk = ep // per

    xp = jnp.pad(x.astype(f32), ((0, np_ - n), (0, 3)))
    xT = xp.T
    W0p = jnp.pad(W0.astype(f32), ((0, 3), (0, 0)))
    b0r = b0.astype(f32).reshape(1, 64)
    z0r, cw = pl.pallas_call(
        _dense_body,
        out_shape=[jax.ShapeDtypeStruct((1, np_), f32),
                   jax.ShapeDtypeStruct((8, 128), f32)],
    )(xT, P.astype(f32), W0p, b0r, W11.astype(f32))
    z0p = z0r.reshape(np_)
    cvec = jnp.broadcast_to(cw[0, 0], (LANE,))

    loop = jnp.arange(n, dtype=jnp.int32)
    pad_e = ep - et
    srcp = jnp.concatenate(
        [edge_index[0].astype(jnp.int32), loop,
         jnp.zeros((pad_e,), jnp.int32)]).reshape(epr, 128)
    dstp = jnp.concatenate(
        [edge_index[1].astype(jnp.int32), loop,
         jnp.zeros((pad_e,), jnp.int32)]).reshape(epr, 128)
    ewp = jnp.concatenate(
        [edge_weight.astype(f32), jnp.ones((n,), f32),
         jnp.zeros((pad_e,), f32)]).reshape(epr, 128)

    mesh = plsc.VectorSubcoreMesh(
        core_axis_name="c", subcore_axis_name="s", num_cores=1)
    sl = np_ // NT
    sc = pl.kernel(
        functools.partial(_sc_body, nchunk, np_),
        out_type=[jax.ShapeDtypeStruct((np_,), f32),
                  jax.ShapeDtypeStruct((epr, 128), f32)],
        mesh=mesh,
        compiler_params=pltpu.CompilerParams(needs_layout_passes=False),
        scratch_types=[
            pltpu.VMEM((np_,), f32),
            pltpu.VMEM((ROWS, 128), jnp.int32),
            pltpu.VMEM((ROWS, 128), jnp.int32),
            pltpu.VMEM((ROWS, 128), f32),
            pltpu.VMEM((ROWS, 128), f32),
            pltpu.VMEM((sl,), f32),
            pltpu.VMEM((sl,), f32),
            pltpu.VMEM((LANE,), f32),
            pltpu.VMEM_SHARED((np_,), f32),
            pltpu.VMEM_SHARED((np_,), f32),
        ],
    )
    zout, _ = sc(srcp, dstp, ewp, z0p, cvec)
    out = zout[:n] + b11[0]
    return out[:, None, None]


# speedup vs baseline: 65.6587x; 1.3921x over previous
import functools

import jax
import jax.numpy as jnp
from jax import lax
from jax.experimental import pallas as pl
from jax.experimental.pallas import tpu as pltpu
from jax.experimental.pallas import tpu_sc as plsc

NT = 16
LANE = 16
CHUNK = 2048
ROWS = CHUNK // 128
T_TERMS = 10
N_ROUNDS = 12


def _dense_body(xT_ref, P_ref, W0p_ref, b0r_ref, W11_ref, z0_ref, cw_ref):
    f32 = jnp.float32
    eye = jnp.eye(64, dtype=f32)
    Wp = eye
    for i in range(P_ref.shape[0]):
        Pi = P_ref[i]
        A = Pi - Pi.T
        W = eye
        term = eye
        for k in range(1, T_TERMS + 1):
            term = jnp.dot(term, A, preferred_element_type=f32) * (1.0 / k)
            W = W + term
        Wp = jnp.dot(Wp, W, preferred_element_type=f32)
    u = jnp.dot(Wp, W11_ref[...], preferred_element_type=f32)
    w8 = jnp.dot(W0p_ref[...], u, preferred_element_type=f32)
    c = jnp.dot(b0r_ref[...], u, preferred_element_type=f32)
    z0_ref[...] = jnp.sum(xT_ref[...] * w8, axis=0, keepdims=True)
    cw_ref[...] = jnp.broadcast_to(c, (8, 128))


def _sc_body(nchunk, np_,
             src_hbm, dst_hbm, ew_hbm, z0_hbm, cvec_hbm,
             zout_hbm, norm_hbm,
             table, srcb, dstb, ewb, msgb, slb, zerob, cbuf,
             lsem0, lsem1, ssem0, ssem1, ssem2, ssem3,
             acc_sh, dinv_sh):
    lsem = (lsem0, lsem1)
    ssem = (ssem0, ssem1, ssem2, ssem3)
    sl = np_ // NT
    wid = lax.axis_index("s")
    base = wid * sl
    row0w = wid * (nchunk * ROWS)

    def zb(i, _):
        zerob[pl.ds(i * LANE, LANE)] = jnp.zeros((LANE,), jnp.float32)
        return 0
    lax.fori_loop(0, sl // LANE, zb, 0)

    pltpu.sync_copy(zerob, acc_sh.at[pl.ds(base, sl)])
    plsc.subcore_barrier()

    def deg_chunk(j, _):
        r0 = row0w + j * ROWS
        pltpu.sync_copy(dst_hbm.at[pl.ds(r0, ROWS)], dstb.at[0])
        pltpu.sync_copy(ew_hbm.at[pl.ds(r0, ROWS)], ewb.at[0])
        for r in range(ROWS):
            pltpu.sync_copy(ewb.at[0, r], acc_sh.at[dstb.at[0, r]], add=True)
        return 0
    lax.fori_loop(0, nchunk, deg_chunk, 0)
    plsc.subcore_barrier()

    pltpu.sync_copy(acc_sh.at[pl.ds(base, sl)], slb)

    def rsq(i, _):
        d = slb[pl.ds(i * LANE, LANE)]
        dc = jnp.maximum(d, 1.0)
        s = 0.5 * (dc + 1.0)
        for _ in range(12):
            s = 0.5 * (s + dc / s)
        y = 1.0 / s
        for _ in range(3):
            y = y * (1.5 - 0.5 * dc * y * y)
        slb[pl.ds(i * LANE, LANE)] = jnp.where(d > 0.0, y, 0.0)
        return 0
    lax.fori_loop(0, sl // LANE, rsq, 0)
    pltpu.sync_copy(slb, dinv_sh.at[pl.ds(base, sl)])
    plsc.subcore_barrier()
    pltpu.sync_copy(dinv_sh, table)

    def norm_chunk(j, _):
        r0 = row0w + j * ROWS
        pltpu.sync_copy(src_hbm.at[pl.ds(r0, ROWS)], srcb.at[0])
        pltpu.sync_copy(dst_hbm.at[pl.ds(r0, ROWS)], dstb.at[0])
        pltpu.sync_copy(ew_hbm.at[pl.ds(r0, ROWS)], ewb.at[0])

        def nv(v, _):
            r = v >> 3
            q = (v & 7) << 4
            si = srcb[0, r, pl.ds(q, LANE)]
            di = dstb[0, r, pl.ds(q, LANE)]
            a = plsc.load_gather(table, [si])
            b = plsc.load_gather(table, [di])
            msgb[0, r, pl.ds(q, LANE)] = a * ewb[0, r, pl.ds(q, LANE)] * b
            return 0
        lax.fori_loop(0, CHUNK // LANE, nv, 0)
        pltpu.sync_copy(msgb.at[0], norm_hbm.at[pl.ds(r0, ROWS)])
        return 0
    lax.fori_loop(0, nchunk, norm_chunk, 0)

    pltpu.sync_copy(z0_hbm, table)
    pltpu.sync_copy(cvec_hbm, cbuf)

    def round_(it, _):
        pltpu.sync_copy(zerob, acc_sh.at[pl.ds(base, sl)])
        plsc.subcore_barrier()

        loads = {}
        scats = {}
        for t in range(nchunk + 1):
            if t < nchunk:
                p4 = t % 4
                p2 = t % 2
                if t >= 4:
                    for d in scats.pop(t - 4):
                        d.wait()
                r0 = row0w + t * ROWS
                loads[t] = [
                    pltpu.async_copy(src_hbm.at[pl.ds(r0, ROWS)],
                                     srcb.at[p2], lsem[p2]),
                    pltpu.async_copy(norm_hbm.at[pl.ds(r0, ROWS)],
                                     ewb.at[p2], lsem[p2]),
                    pltpu.async_copy(dst_hbm.at[pl.ds(r0, ROWS)],
                                     dstb.at[p4], lsem[p2]),
                ]
            if t >= 1:
                j = t - 1
                for d in loads.pop(j):
                    d.wait()
                jp2 = j % 2
                jp4 = j % 4

                def gv(i, _, jp2=jp2, jp4=jp4):
                    v = i * 4
                    for u in range(4):
                        r = (v + u) >> 3
                        q = ((v + u) & 7) << 4
                        si = srcb[jp2, r, pl.ds(q, LANE)]
                        zv = plsc.load_gather(table, [si])
                        msgb[jp4, r, pl.ds(q, LANE)] = (
                            zv * ewb[jp2, r, pl.ds(q, LANE)])
                    return 0
                lax.fori_loop(0, (CHUNK // LANE) // 4, gv, 0)
                for r in range(ROWS):
                    pltpu.sync_copy(msgb.at[jp4, r],
                                    acc_sh.at[dstb.at[jp4, r]], add=True)
                scats[j] = []
        for j in sorted(scats):
            for d in scats[j]:
                d.wait()
        plsc.subcore_barrier()

        @pl.when(wid == 0)
        def _flush():
            pltpu.sync_copy(acc_sh, zout_hbm)
        plsc.subcore_barrier()
        pltpu.sync_copy(zout_hbm, table)

        @pl.when(it == 0)
        def _add_c():
            cval = cbuf[...]

            def ac(i, _):
                tv = table[pl.ds(i * LANE, LANE)]
                table[pl.ds(i * LANE, LANE)] = tv + cval
                return 0
            lax.fori_loop(0, np_ // LANE, ac, 0)
        return 0
    lax.fori_loop(0, N_ROUNDS, round_, 0)


@jax.jit
def kernel(x, edge_index, edge_weight, W0, b0, P, W11, b11):
    f32 = jnp.float32
    n = x.shape[0]
    e = edge_index.shape[1]
    np_ = ((n + 255) // 256) * 256
    et = e + n
    per = NT * CHUNK
    ep = ((et + per - 1) // per) * per
    epr = ep // 128
    nchunk = ep // per

    xp = jnp.pad(x.astype(f32), ((0, np_ - n), (0, 3)))
    xT = xp.T
    W0p = jnp.pad(W0.astype(f32), ((0, 3), (0, 0)))
    b0r = b0.astype(f32).reshape(1, 64)
    z0r, cw = pl.pallas_call(
        _dense_body,
        out_shape=[jax.ShapeDtypeStruct((1, np_), f32),
                   jax.ShapeDtypeStruct((8, 128), f32)],
    )(xT, P.astype(f32), W0p, b0r, W11.astype(f32))
    z0p = z0r.reshape(np_)
    cvec = jnp.broadcast_to(cw[0, 0], (LANE,))

    loop = jnp.arange(n, dtype=jnp.int32)
    pad_e = ep - et
    srcp = jnp.concatenate(
        [edge_index[0].astype(jnp.int32), loop,
         jnp.zeros((pad_e,), jnp.int32)]).reshape(epr, 128)
    dstp = jnp.concatenate(
        [edge_index[1].astype(jnp.int32), loop,
         jnp.zeros((pad_e,), jnp.int32)]).reshape(epr, 128)
    ewp = jnp.concatenate(
        [edge_weight.astype(f32), jnp.ones((n,), f32),
         jnp.zeros((pad_e,), f32)]).reshape(epr, 128)

    mesh = plsc.VectorSubcoreMesh(
        core_axis_name="c", subcore_axis_name="s", num_cores=1)
    sl = np_ // NT
    sc = pl.kernel(
        functools.partial(_sc_body, nchunk, np_),
        out_type=[jax.ShapeDtypeStruct((np_,), f32),
                  jax.ShapeDtypeStruct((epr, 128), f32)],
        mesh=mesh,
        compiler_params=pltpu.CompilerParams(needs_layout_passes=False),
        scratch_types=[
            pltpu.VMEM((np_,), f32),
            pltpu.VMEM((2, ROWS, 128), jnp.int32),
            pltpu.VMEM((4, ROWS, 128), jnp.int32),
            pltpu.VMEM((2, ROWS, 128), f32),
            pltpu.VMEM((4, ROWS, 128), f32),
            pltpu.VMEM((sl,), f32),
            pltpu.VMEM((sl,), f32),
            pltpu.VMEM((LANE,), f32),
            pltpu.SemaphoreType.DMA,
            pltpu.SemaphoreType.DMA,
            pltpu.SemaphoreType.DMA,
            pltpu.SemaphoreType.DMA,
            pltpu.SemaphoreType.DMA,
            pltpu.SemaphoreType.DMA,
            pltpu.VMEM_SHARED((np_,), f32),
            pltpu.VMEM_SHARED((np_,), f32),
        ],
    )
    zout, _ = sc(srcp, dstp, ewp, z0p, cvec)
    out = zout[:n] + b11[0]
    return out[:, None, None]


# speedup vs baseline: 105.7195x; 1.6101x over previous
import functools

import jax
import jax.numpy as jnp
from jax import lax
from jax.experimental import pallas as pl
from jax.experimental.pallas import tpu as pltpu
from jax.experimental.pallas import tpu_sc as plsc

NT = 16
LANE = 16
CHUNK = 2048
ROWS = CHUNK // 128
T_TERMS = 10
N_ROUNDS = 12


def _dense_body(xT_ref, P_ref, W0p_ref, b0r_ref, W11_ref, z0_ref, cw_ref):
    f32 = jnp.float32
    eye = jnp.eye(64, dtype=f32)
    Wp = eye
    for i in range(P_ref.shape[0]):
        Pi = P_ref[i]
        A = Pi - Pi.T
        W = eye
        term = eye
        for k in range(1, T_TERMS + 1):
            term = jnp.dot(term, A, preferred_element_type=f32) * (1.0 / k)
            W = W + term
        Wp = jnp.dot(Wp, W, preferred_element_type=f32)
    u = jnp.dot(Wp, W11_ref[...], preferred_element_type=f32)
    w8 = jnp.dot(W0p_ref[...], u, preferred_element_type=f32)
    c = jnp.dot(b0r_ref[...], u, preferred_element_type=f32)
    z0_ref[...] = jnp.sum(xT_ref[...] * w8, axis=0, keepdims=True)
    cw_ref[...] = jnp.broadcast_to(c, (8, 128))


def _sc_body(nchunk, np_,
             src_hbm, dst_hbm, ew_hbm, z0_hbm, cvec_hbm,
             zout_hbm, norm_hbm,
             table, srcb, dstb, ewb, msgb, slb, zerob, cbuf,
             lsem0, lsem1, ssem0, ssem1, ssem2, ssem3,
             acc_sh, dinv_sh):
    lsem = (lsem0, lsem1)
    ssem = (ssem0, ssem1, ssem2, ssem3)
    sl = np_ // NT
    wid = lax.axis_index("s")
    base = wid * sl
    cw0 = wid * nchunk
    row0w = wid * (nchunk * ROWS)

    def zb(i, _):
        zerob[pl.ds(i * LANE, LANE)] = jnp.zeros((LANE,), jnp.float32)
        return 0
    lax.fori_loop(0, sl // LANE, zb, 0)

    pltpu.sync_copy(zerob, acc_sh.at[pl.ds(base, sl)])
    plsc.subcore_barrier()

    def deg_chunk(j, _):
        r0 = row0w + j * ROWS
        pltpu.sync_copy(dst_hbm.at[pl.ds(r0, ROWS)], dstb.at[0])
        pltpu.sync_copy(ew_hbm.at[pl.ds(r0, ROWS)], ewb.at[0])
        for r in range(ROWS):
            pltpu.sync_copy(ewb.at[0, r], acc_sh.at[dstb.at[0, r]], add=True)
        return 0
    lax.fori_loop(0, nchunk, deg_chunk, 0)
    plsc.subcore_barrier()

    pltpu.sync_copy(acc_sh.at[pl.ds(base, sl)], slb)

    def rsq(i, _):
        d = slb[pl.ds(i * LANE, LANE)]
        dc = jnp.maximum(d, 1.0)
        s = 0.5 * (dc + 1.0)
        for _ in range(12):
            s = 0.5 * (s + dc / s)
        y = 1.0 / s
        for _ in range(3):
            y = y * (1.5 - 0.5 * dc * y * y)
        slb[pl.ds(i * LANE, LANE)] = jnp.where(d > 0.0, y, 0.0)
        return 0
    lax.fori_loop(0, sl // LANE, rsq, 0)
    pltpu.sync_copy(slb, dinv_sh.at[pl.ds(base, sl)])
    plsc.subcore_barrier()
    pltpu.sync_copy(dinv_sh, table)

    def norm_chunk(j, _):
        r0 = row0w + j * ROWS
        pltpu.sync_copy(src_hbm.at[pl.ds(r0, ROWS)], srcb.at[0])
        pltpu.sync_copy(dst_hbm.at[pl.ds(r0, ROWS)], dstb.at[0])
        pltpu.sync_copy(ew_hbm.at[pl.ds(r0, ROWS)], ewb.at[0])

        def nv(v, _):
            r = v >> 3
            q = (v & 7) << 4
            si = srcb[0, r, pl.ds(q, LANE)]
            di = dstb[0, r, pl.ds(q, LANE)]
            a = plsc.load_gather(table, [si])
            b = plsc.load_gather(table, [di])
            msgb[0, r, pl.ds(q, LANE)] = a * ewb[0, r, pl.ds(q, LANE)] * b
            return 0
        lax.fori_loop(0, CHUNK // LANE, nv, 0)
        pltpu.sync_copy(msgb.at[0], norm_hbm.at[pl.ds(r0, ROWS)])
        return 0
    lax.fori_loop(0, nchunk, norm_chunk, 0)

    pltpu.sync_copy(z0_hbm, table)
    pltpu.sync_copy(cvec_hbm, cbuf)

    def round_(it, _):
        pltpu.sync_copy(zerob, acc_sh.at[pl.ds(base, sl)])
        plsc.subcore_barrier()

        loads = {}
        scats = {}
        for t in range(nchunk + 1):
            if t < nchunk:
                p4 = t % 4
                p2 = t % 2
                r0 = row0w + t * ROWS
                loads[t] = [
                    pltpu.async_copy(src_hbm.at[pl.ds(r0, ROWS)],
                                     srcb.at[p2], lsem[p2]),
                    pltpu.async_copy(norm_hbm.at[pl.ds(r0, ROWS)],
                                     ewb.at[p2], lsem[p2]),
                    pltpu.async_copy(dst_hbm.at[pl.ds(r0, ROWS)],
                                     dstb.at[p4], lsem[p2]),
                ]
            if t >= 1:
                j = t - 1
                for d in loads.pop(j):
                    d.wait()
                jp2 = j % 2
                jp4 = j % 4

                def gv(i, _, jp2=jp2, jp4=jp4):
                    v = i * 4
                    for u in range(4):
                        r = (v + u) >> 3
                        q = ((v + u) & 7) << 4
                        si = srcb[jp2, r, pl.ds(q, LANE)]
                        zv = plsc.load_gather(table, [si])
                        msgb[jp4, r, pl.ds(q, LANE)] = (
                            zv * ewb[jp2, r, pl.ds(q, LANE)])
                    return 0
                lax.fori_loop(0, (CHUNK // LANE) // 4, gv, 0)
                if j >= 1:
                    for d in scats.pop(j - 1):
                        d.wait()
                scats[j] = [
                    pltpu.async_copy(msgb.at[jp4, r],
                                     acc_sh.at[dstb.at[jp4, r]],
                                     ssem[jp4], add=True)
                    for r in range(ROWS)
                ]
        for j in sorted(scats):
            for d in scats[j]:
                d.wait()
        plsc.subcore_barrier()

        @pl.when(wid == 0)
        def _flush():
            pltpu.sync_copy(acc_sh, zout_hbm)
        plsc.subcore_barrier()
        pltpu.sync_copy(zout_hbm, table)

        @pl.when(it == 0)
        def _add_c():
            cval = cbuf[...]

            def ac(i, _):
                tv = table[pl.ds(i * LANE, LANE)]
                table[pl.ds(i * LANE, LANE)] = tv + cval
                return 0
            lax.fori_loop(0, np_ // LANE, ac, 0)
        return 0
    lax.fori_loop(0, N_ROUNDS, round_, 0)


@jax.jit
def kernel(x, edge_index, edge_weight, W0, b0, P, W11, b11):
    f32 = jnp.float32
    n = x.shape[0]
    e = edge_index.shape[1]
    np_ = ((n + 255) // 256) * 256
    et = e + n
    per = NT * CHUNK
    ep = ((et + per - 1) // per) * per
    epr = ep // 128
    nchunk = ep // per

    xp = jnp.pad(x.astype(f32), ((0, np_ - n), (0, 3)))
    xT = xp.T
    W0p = jnp.pad(W0.astype(f32), ((0, 3), (0, 0)))
    b0r = b0.astype(f32).reshape(1, 64)
    z0r, cw = pl.pallas_call(
        _dense_body,
        out_shape=[jax.ShapeDtypeStruct((1, np_), f32),
                   jax.ShapeDtypeStruct((8, 128), f32)],
    )(xT, P.astype(f32), W0p, b0r, W11.astype(f32))
    z0p = z0r.reshape(np_)
    cvec = jnp.broadcast_to(cw[0, 0], (LANE,))

    loop = jnp.arange(n, dtype=jnp.int32)
    pad_e = ep - et
    srcp = jnp.concatenate(
        [edge_index[0].astype(jnp.int32), loop,
         jnp.zeros((pad_e,), jnp.int32)]).reshape(ep // 128, 128)
    dstp = jnp.concatenate(
        [edge_index[1].astype(jnp.int32), loop,
         jnp.zeros((pad_e,), jnp.int32)]).reshape(ep // 128, 128)
    ewp = jnp.concatenate(
        [edge_weight.astype(f32), jnp.ones((n,), f32),
         jnp.zeros((pad_e,), f32)]).reshape(ep // 128, 128)

    mesh = plsc.VectorSubcoreMesh(
        core_axis_name="c", subcore_axis_name="s", num_cores=1)
    sl = np_ // NT
    sc = pl.kernel(
        functools.partial(_sc_body, nchunk, np_),
        out_type=[jax.ShapeDtypeStruct((np_,), f32),
                  jax.ShapeDtypeStruct((ep // 128, 128), f32)],
        mesh=mesh,
        compiler_params=pltpu.CompilerParams(needs_layout_passes=False),
        scratch_types=[
            pltpu.VMEM((np_,), f32),
            pltpu.VMEM((2, ROWS, 128), jnp.int32),
            pltpu.VMEM((4, ROWS, 128), jnp.int32),
            pltpu.VMEM((2, ROWS, 128), f32),
            pltpu.VMEM((4, ROWS, 128), f32),
            pltpu.VMEM((sl,), f32),
            pltpu.VMEM((sl,), f32),
            pltpu.VMEM((LANE,), f32),
            pltpu.SemaphoreType.DMA,
            pltpu.SemaphoreType.DMA,
            pltpu.SemaphoreType.DMA,
            pltpu.SemaphoreType.DMA,
            pltpu.SemaphoreType.DMA,
            pltpu.SemaphoreType.DMA,
            pltpu.VMEM_SHARED((np_,), f32),
            pltpu.VMEM_SHARED((np_,), f32),
        ],
    )
    zout, _ = sc(srcp, dstp, ewp, z0p, cvec)
    out = zout[:n] + b11[0]
    return out[:, None, None]


# speedup vs baseline: 116.3278x; 1.1003x over previous
import functools

import jax
import jax.numpy as jnp
from jax import lax
from jax.experimental import pallas as pl
from jax.experimental.pallas import tpu as pltpu
from jax.experimental.pallas import tpu_sc as plsc

NT = 16
LANE = 16
CHUNK = 1024
ROWS = CHUNK // 128
T_TERMS = 10
N_ROUNDS = 12


def _dense_body(xT_ref, P_ref, W0p_ref, b0r_ref, W11_ref, z0_ref, cw_ref):
    f32 = jnp.float32
    eye = jnp.eye(64, dtype=f32)
    Wp = eye
    for i in range(P_ref.shape[0]):
        Pi = P_ref[i]
        A = Pi - Pi.T
        W = eye
        term = eye
        for k in range(1, T_TERMS + 1):
            term = jnp.dot(term, A, preferred_element_type=f32) * (1.0 / k)
            W = W + term
        Wp = jnp.dot(Wp, W, preferred_element_type=f32)
    u = jnp.dot(Wp, W11_ref[...], preferred_element_type=f32)
    w8 = jnp.dot(W0p_ref[...], u, preferred_element_type=f32)
    c = jnp.dot(b0r_ref[...], u, preferred_element_type=f32)
    z0_ref[...] = jnp.sum(xT_ref[...] * w8, axis=0, keepdims=True)
    cw_ref[...] = jnp.broadcast_to(c, (8, 128))


def _sc_body(nchunk, np_,
             src_hbm, dst_hbm, ew_hbm, z0_hbm, cvec_hbm,
             zout_hbm, norm_hbm,
             table, srcb, dstb, ewb, msgb, slb, zerob, cbuf,
             ls0, ls1, ls2, ls3, ss0, ss1, ss2, ss3,
             acc_sh, dinv_sh):
    lsem = (ls0, ls1, ls2, ls3)
    ssem = (ss0, ss1, ss2, ss3)
    sl = np_ // NT
    wid = lax.axis_index("s")
    base = wid * sl
    row0w = wid * (nchunk * ROWS)
    nb = nchunk // 4

    def zb(i, _):
        zerob[pl.ds(i * LANE, LANE)] = jnp.zeros((LANE,), jnp.float32)
        return 0
    lax.fori_loop(0, sl // LANE, zb, 0)

    def walk(mode):
        def drain(k):
            if mode == 1:
                pltpu.make_async_copy(
                    msgb.at[k], norm_hbm.at[pl.ds(row0w, ROWS)],
                    ssem[k]).wait()
            else:
                srcbuf = ewb if mode == 0 else msgb
                for r in range(ROWS):
                    pltpu.make_async_copy(
                        srcbuf.at[k, r], acc_sh.at[dstb.at[k, r]],
                        ssem[k]).wait()

        def body(b, _):
            c0 = b * 4
            for k in range(4):
                @pl.when(b > 0)
                def _d(k=k):
                    drain(k)
                r0 = row0w + (c0 + k) * ROWS
                pltpu.async_copy(dst_hbm.at[pl.ds(r0, ROWS)],
                                 dstb.at[k], lsem[k])
                if mode == 0:
                    pltpu.async_copy(ew_hbm.at[pl.ds(r0, ROWS)],
                                     ewb.at[k], lsem[k])
                elif mode == 1:
                    pltpu.async_copy(src_hbm.at[pl.ds(r0, ROWS)],
                                     srcb.at[k], lsem[k])
                    pltpu.async_copy(ew_hbm.at[pl.ds(r0, ROWS)],
                                     ewb.at[k], lsem[k])
                else:
                    pltpu.async_copy(src_hbm.at[pl.ds(r0, ROWS)],
                                     srcb.at[k], lsem[k])
                    pltpu.async_copy(norm_hbm.at[pl.ds(r0, ROWS)],
                                     ewb.at[k], lsem[k])
            for k in range(4):
                r0 = row0w + (c0 + k) * ROWS
                nld = 2 if mode == 0 else 3
                for _ in range(nld):
                    pltpu.make_async_copy(dst_hbm.at[pl.ds(r0, ROWS)],
                                          dstb.at[k], lsem[k]).wait()
                if mode == 1:
                    def nv(i, _, k=k):
                        v = i * 8
                        for u in range(8):
                            r = (v + u) >> 3
                            q = ((v + u) & 7) << 4
                            si = srcb[k, r, pl.ds(q, LANE)]
                            di = dstb[k, r, pl.ds(q, LANE)]
                            a = plsc.load_gather(table, [si])
                            bb = plsc.load_gather(table, [di])
                            msgb[k, r, pl.ds(q, LANE)] = (
                                a * ewb[k, r, pl.ds(q, LANE)] * bb)
                        return 0
                    lax.fori_loop(0, (CHUNK // LANE) // 8, nv, 0)
                    pltpu.async_copy(msgb.at[k],
                                     norm_hbm.at[pl.ds(r0, ROWS)], ssem[k])
                elif mode == 2:
                    def gv(i, _, k=k):
                        v = i * 8
                        for u in range(8):
                            r = (v + u) >> 3
                            q = ((v + u) & 7) << 4
                            si = srcb[k, r, pl.ds(q, LANE)]
                            zv = plsc.load_gather(table, [si])
                            msgb[k, r, pl.ds(q, LANE)] = (
                                zv * ewb[k, r, pl.ds(q, LANE)])
                        return 0
                    lax.fori_loop(0, (CHUNK // LANE) // 8, gv, 0)
                    for r in range(ROWS):
                        pltpu.async_copy(msgb.at[k, r],
                                         acc_sh.at[dstb.at[k, r]],
                                         ssem[k], add=True)
                else:
                    for r in range(ROWS):
                        pltpu.async_copy(ewb.at[k, r],
                                         acc_sh.at[dstb.at[k, r]],
                                         ssem[k], add=True)
            return 0
        lax.fori_loop(0, nb, body, 0)
        for k in range(4):
            drain(k)

    pltpu.sync_copy(zerob, acc_sh.at[pl.ds(base, sl)])
    plsc.subcore_barrier()
    walk(0)
    plsc.subcore_barrier()

    pltpu.sync_copy(acc_sh.at[pl.ds(base, sl)], slb)

    def rsq(i, _):
        d = slb[pl.ds(i * LANE, LANE)]
        dc = jnp.maximum(d, 1.0)
        s = 0.5 * (dc + 1.0)
        for _ in range(12):
            s = 0.5 * (s + dc / s)
        y = 1.0 / s
        for _ in range(3):
            y = y * (1.5 - 0.5 * dc * y * y)
        slb[pl.ds(i * LANE, LANE)] = jnp.where(d > 0.0, y, 0.0)
        return 0
    lax.fori_loop(0, sl // LANE, rsq, 0)
    pltpu.sync_copy(slb, dinv_sh.at[pl.ds(base, sl)])
    plsc.subcore_barrier()
    pltpu.sync_copy(dinv_sh, table)

    walk(1)

    pltpu.sync_copy(z0_hbm, table)
    pltpu.sync_copy(cvec_hbm, cbuf)

    def round_(it, _):
        pltpu.sync_copy(zerob, acc_sh.at[pl.ds(base, sl)])
        plsc.subcore_barrier()
        walk(2)
        plsc.subcore_barrier()

        @pl.when(wid == 0)
        def _flush():
            pltpu.sync_copy(acc_sh, zout_hbm)
        plsc.subcore_barrier()
        pltpu.sync_copy(zout_hbm, table)

        @pl.when(it == 0)
        def _add_c():
            cval = cbuf[...]

            def ac(i, _):
                tv = table[pl.ds(i * LANE, LANE)]
                table[pl.ds(i * LANE, LANE)] = tv + cval
                return 0
            lax.fori_loop(0, np_ // LANE, ac, 0)
        return 0
    lax.fori_loop(0, N_ROUNDS, round_, 0)


@jax.jit
def kernel(x, edge_index, edge_weight, W0, b0, P, W11, b11):
    f32 = jnp.float32
    n = x.shape[0]
    e = edge_index.shape[1]
    np_ = ((n + 255) // 256) * 256
    et = e + n
    per = NT * CHUNK
    ep = ((et + per - 1) // per) * per
    epr = ep // 128
    nchunk = ep // per

    xp = jnp.pad(x.astype(f32), ((0, np_ - n), (0, 3)))
    xT = xp.T
    W0p = jnp.pad(W0.astype(f32), ((0, 3), (0, 0)))
    b0r = b0.astype(f32).reshape(1, 64)
    z0r, cw = pl.pallas_call(
        _dense_body,
        out_shape=[jax.ShapeDtypeStruct((1, np_), f32),
                   jax.ShapeDtypeStruct((8, 128), f32)],
    )(xT, P.astype(f32), W0p, b0r, W11.astype(f32))
    z0p = z0r.reshape(np_)
    cvec = jnp.broadcast_to(cw[0, 0], (LANE,))

    loop = jnp.arange(n, dtype=jnp.int32)
    pad_e = ep - et
    srcp = jnp.concatenate(
        [edge_index[0].astype(jnp.int32), loop,
         jnp.zeros((pad_e,), jnp.int32)]).reshape(ep // 128, 128)
    dstp = jnp.concatenate(
        [edge_index[1].astype(jnp.int32), loop,
         jnp.zeros((pad_e,), jnp.int32)]).reshape(ep // 128, 128)
    ewp = jnp.concatenate(
        [edge_weight.astype(f32), jnp.ones((n,), f32),
         jnp.zeros((pad_e,), f32)]).reshape(ep // 128, 128)

    mesh = plsc.VectorSubcoreMesh(
        core_axis_name="c", subcore_axis_name="s", num_cores=1)
    sl = np_ // NT
    sc = pl.kernel(
        functools.partial(_sc_body, nchunk, np_),
        out_type=[jax.ShapeDtypeStruct((np_,), f32),
                  jax.ShapeDtypeStruct((ep // 128, 128), f32)],
        mesh=mesh,
        compiler_params=pltpu.CompilerParams(needs_layout_passes=False),
        scratch_types=[
            pltpu.VMEM((np_,), f32),
            pltpu.VMEM((4, ROWS, 128), jnp.int32),
            pltpu.VMEM((4, ROWS, 128), jnp.int32),
            pltpu.VMEM((4, ROWS, 128), f32),
            pltpu.VMEM((4, ROWS, 128), f32),
            pltpu.VMEM((sl,), f32),
            pltpu.VMEM((sl,), f32),
            pltpu.VMEM((LANE,), f32),
            pltpu.SemaphoreType.DMA,
            pltpu.SemaphoreType.DMA,
            pltpu.SemaphoreType.DMA,
            pltpu.SemaphoreType.DMA,
            pltpu.SemaphoreType.DMA,
            pltpu.SemaphoreType.DMA,
            pltpu.SemaphoreType.DMA,
            pltpu.SemaphoreType.DMA,
            pltpu.VMEM_SHARED((np_,), f32),
            pltpu.VMEM_SHARED((np_,), f32),
        ],
    )
    zout, _ = sc(srcp, dstp, ewp, z0p, cvec)
    out = zout[:n] + b11[0]
    return out[:, None, None]
